# baseline (speedup 1.0000x reference)
import functools

import jax
import jax.numpy as jnp
from jax import lax
from jax.experimental import pallas as pl
from jax.experimental.pallas import tpu as pltpu
from jax.experimental.pallas import tpu_sc as plsc

V = 4
D = 100
DP = 128

R, C0 = 16384, 200
B = R * C0

NC, NS = 2, 16
NW = NC * NS
B_PER_W = B // NW
CHUNK = 128
N_CHUNKS = B_PER_W // CHUNK


def _sc_gather_body(stable_hbm, x_hbm, out_hbm,
                    idx_a, idx_b, rows_a, rows_b, sem_a, sem_b):
    wid = lax.axis_index("s") * NC + lax.axis_index("c")
    base = wid * B_PER_W
    bufs = ((idx_a, rows_a, sem_a), (idx_b, rows_b, sem_b))

    def issue(j, idx, rows, sem):
        off = base + j * CHUNK
        pltpu.sync_copy(x_hbm.at[pl.ds(off, CHUNK)], idx)
        pltpu.async_copy(stable_hbm.at[idx], rows, sem)

    issue(0, *bufs[0])

    def step(j, carry):
        for p in (0, 1):
            @pl.when(j % 2 == p)
            def _():
                idx, rows, sem = bufs[p]
                pltpu.make_async_copy(stable_hbm.at[idx], rows, sem).wait()
                oidx, orows, osem = bufs[1 - p]

                @pl.when(j + 1 < N_CHUNKS)
                def _():
                    issue(j + 1, oidx, orows, osem)

                pltpu.sync_copy(rows, out_hbm.at[pl.ds(base + j * CHUNK, CHUNK)])
        return carry

    lax.fori_loop(0, N_CHUNKS, step, 0)


_sc_gather = functools.partial(
    pl.kernel,
    out_type=jax.ShapeDtypeStruct((B, DP), jnp.float32),
    mesh=plsc.VectorSubcoreMesh(core_axis_name="c", subcore_axis_name="s"),
    scratch_types=[
        pltpu.VMEM((CHUNK,), jnp.int32),
        pltpu.VMEM((CHUNK,), jnp.int32),
        pltpu.VMEM((CHUNK, DP), jnp.float32),
        pltpu.VMEM((CHUNK, DP), jnp.float32),
        pltpu.SemaphoreType.DMA,
        pltpu.SemaphoreType.DMA,
    ],
)(_sc_gather_body)


def kernel(x, table):
    xf = x.reshape(B).astype(jnp.int32)
    stable = jnp.zeros((V, DP), jnp.float32).at[:, :D].set(table.astype(jnp.float32))
    padded = _sc_gather(stable, xf)
    return padded[:, :D].reshape(R, C0, D)


# speedup vs baseline: 86.1039x; 86.1039x over previous
import jax
import jax.numpy as jnp
from jax.experimental import pallas as pl

V = 4
D = 100
R, C0 = 16384, 200

BJ = 8
BI = 8192
DD = 10
GJ = C0 // BJ
GI = R // BI


def _select_body(xt_ref, table_ref, out_ref):
    xb = xt_ref[...][None, :, :]
    t = table_ref[...]
    is1 = xb == 1
    is3 = xb == 3
    islo = xb < 2
    for d0 in range(0, D, DD):
        c0 = t[0, d0:d0 + DD][:, None, None]
        c1 = t[1, d0:d0 + DD][:, None, None]
        c2 = t[2, d0:d0 + DD][:, None, None]
        c3 = t[3, d0:d0 + DD][:, None, None]
        lo = jnp.where(is1, c1, c0)
        hi = jnp.where(is3, c3, c2)
        out_ref[d0:d0 + DD] = jnp.where(islo, lo, hi)


def kernel(x, table):
    xt = jnp.transpose(x.astype(jnp.int32), (1, 0))
    o3 = pl.pallas_call(
        _select_body,
        grid=(GJ, GI),
        in_specs=[
            pl.BlockSpec((BJ, BI), lambda j, i: (j, i)),
            pl.BlockSpec((V, D), lambda j, i: (0, 0)),
        ],
        out_specs=pl.BlockSpec((D, BJ, BI), lambda j, i: (0, j, i)),
        out_shape=jax.ShapeDtypeStruct((D, C0, R), jnp.float32),
    )(xt, table.astype(jnp.float32))
    return jnp.transpose(o3, (2, 1, 0))
